# baseline (speedup 1.0000x reference)
import jax
import jax.numpy as jnp
from jax.experimental import pallas as pl
from jax.experimental.pallas import tpu as pltpu

_N = 2048
_D = 128
_P = 8
_TOPK = 32
_PD = _P * _D
_BLK = 256
_GRID = _N // _BLK
_BISECT_ITERS = 21

_PARALLEL = pltpu.CompilerParams(dimension_semantics=("parallel",))


def _prep_kernel(f_ref, w_ref, y_ref, yt_ref):
    f = f_ref[...]
    w = w_ref[...]
    cols = []
    for p in range(_P):
        fw = f * w[p:p + 1, :]
        n = jnp.sqrt(jnp.sum(fw * fw, axis=1, keepdims=True))
        cols.append(fw / jnp.maximum(n, 1e-12))
    y = jnp.concatenate(cols, axis=1)
    y_ref[...] = y
    yt_ref[...] = y.T


def _sim_kernel(y_ref, yt_ref, s_ref, t_ref):
    s = jax.lax.dot_general(
        y_ref[...], yt_ref[...], (((1,), (0,)), ((), ())),
        preferred_element_type=jnp.float32) * (1.0 / _P)
    s_ref[...] = s

    def body(_, carry):
        lo, hi = carry
        mid = (lo + hi) * 0.5
        cnt = jnp.count_nonzero(s >= mid, axis=1, keepdims=True)
        pred = cnt >= _TOPK
        return jnp.where(pred, mid, lo), jnp.where(pred, hi, mid)

    lo, _ = jax.lax.fori_loop(
        0, _BISECT_ITERS, body,
        (jnp.full((_BLK, 1), -1.25, jnp.float32),
         jnp.full((_BLK, 1), 1.25, jnp.float32)))
    t_ref[...] = lo


def _mask_kernel(s_ref, tc_ref, tr_ref, o_ref):
    s = s_ref[...]
    ti = tc_ref[...]
    tj = tr_ref[...]
    keep = (s >= ti).astype(jnp.float32) + (s >= tj).astype(jnp.float32)
    o_ref[...] = s * keep * 0.5


@jax.jit
def kernel(features, weight_tensor):
    y, yt = pl.pallas_call(
        _prep_kernel,
        grid=(_GRID,),
        in_specs=[
            pl.BlockSpec((_BLK, _D), lambda i: (i, 0)),
            pl.BlockSpec((_P, _D), lambda i: (0, 0)),
        ],
        out_specs=[
            pl.BlockSpec((_BLK, _PD), lambda i: (i, 0)),
            pl.BlockSpec((_PD, _BLK), lambda i: (0, i)),
        ],
        out_shape=[
            jax.ShapeDtypeStruct((_N, _PD), jnp.float32),
            jax.ShapeDtypeStruct((_PD, _N), jnp.float32),
        ],
        compiler_params=_PARALLEL,
    )(features, weight_tensor)

    s, tcol = pl.pallas_call(
        _sim_kernel,
        grid=(_GRID,),
        in_specs=[
            pl.BlockSpec((_BLK, _PD), lambda i: (i, 0)),
            pl.BlockSpec((_PD, _N), lambda i: (0, 0)),
        ],
        out_specs=[
            pl.BlockSpec((_BLK, _N), lambda i: (i, 0)),
            pl.BlockSpec((_BLK, 1), lambda i: (i, 0)),
        ],
        out_shape=[
            jax.ShapeDtypeStruct((_N, _N), jnp.float32),
            jax.ShapeDtypeStruct((_N, 1), jnp.float32),
        ],
        compiler_params=_PARALLEL,
    )(y, yt)

    trow = tcol.reshape(1, _N)

    out = pl.pallas_call(
        _mask_kernel,
        grid=(_GRID,),
        in_specs=[
            pl.BlockSpec((_BLK, _N), lambda i: (i, 0)),
            pl.BlockSpec((_BLK, 1), lambda i: (i, 0)),
            pl.BlockSpec((1, _N), lambda i: (0, 0)),
        ],
        out_specs=pl.BlockSpec((_BLK, _N), lambda i: (i, 0)),
        out_shape=jax.ShapeDtypeStruct((_N, _N), jnp.float32),
        compiler_params=_PARALLEL,
    )(s, tcol, trow)
    return out
